# baseline (speedup 1.0000x reference)
import functools

import jax
import jax.numpy as jnp
from jax import lax
from jax.experimental import pallas as pl
from jax.experimental.pallas import tpu as pltpu
from jax.experimental.pallas import tpu_sc as plsc

NUM_CORES = 2
NUM_SUBCORES = 16
NUM_WORKERS = NUM_CORES * NUM_SUBCORES

EMB_DIM = 32
PACK = 128 // EMB_DIM
BURST = 128
OUT_ROWS = BURST * EMB_DIM // 128


def _gather_kernel(n_bursts):
    tokens_per_w = n_bursts * BURST
    out_rows_w = n_bursts * OUT_ROWS

    mesh = plsc.VectorSubcoreMesh(core_axis_name="c", subcore_axis_name="s")

    @functools.partial(
        pl.kernel,
        mesh=mesh,
        out_type=jax.ShapeDtypeStruct(
            (NUM_WORKERS * n_bursts, OUT_ROWS, 128), jnp.float32),
        scratch_types=[
            pltpu.VMEM((n_bursts, BURST), jnp.int32),
            pltpu.VMEM((n_bursts, BURST), jnp.int32),
            pltpu.VMEM((BURST, 128), jnp.float32),
            pltpu.VMEM((BURST, 128), jnp.float32),
            pltpu.VMEM((OUT_ROWS, 128), jnp.float32),
            pltpu.VMEM((OUT_ROWS, 128), jnp.float32),
            pltpu.SemaphoreType.DMA,
            pltpu.SemaphoreType.DMA,
        ],
    )
    def body(idx_hbm, table_hbm, out_hbm, idx_v, q_v, st0, st1, ob0, ob1,
             gsem, wsem):
        wid = lax.axis_index("s") * NUM_CORES + lax.axis_index("c")
        pltpu.sync_copy(idx_hbm.at[pl.ds(wid * n_bursts, n_bursts)], idx_v)
        out_base = wid * n_bursts

        def meta_row(j, carry):
            for k in range(BURST // 16):
                q_v[j, pl.ds(k * 16, 16)] = lax.shift_right_logical(
                    idx_v[j, pl.ds(k * 16, 16)], PACK // 2)
            return carry

        lax.fori_loop(0, n_bursts, meta_row, 0)

        def gather_desc(blk, stage):
            return pltpu.make_async_copy(table_hbm.at[q_v.at[blk]], stage,
                                         gsem)

        def write_desc(blk, ob):
            return pltpu.make_async_copy(ob, out_hbm.at[out_base + blk],
                                         wsem)

        def select(blk, stage, ob):
            def sel_grp(g, carry):
                tok_vec = idx_v[blk, pl.ds(g * 16, 16)]
                for l in range(16):
                    n = g * 16 + l
                    src = (tok_vec[l] & (PACK - 1)) * EMB_DIM
                    dst = (l & (PACK - 1)) * EMB_DIM
                    orow = g * 4 + l // 4
                    for h in range(EMB_DIM // 16):
                        ob[orow, pl.ds(dst + h * 16, 16)] = (
                            stage[n, pl.ds(src + h * 16, 16)])
                return carry

            lax.fori_loop(0, BURST // 16, sel_grp, 0)

        gather_desc(0, st0).start()

        def step(i, carry):
            for b, (cur, nxt, ocur, onxt) in ((0, (st0, st1, ob0, ob1)),
                                              (1, (st1, st0, ob1, ob0))):
                blk = 2 * i + b

                @pl.when(blk >= 1)
                def _():
                    write_desc(blk - 1, onxt).wait()

                @pl.when(blk + 1 < n_bursts)
                def _():
                    gather_desc(blk + 1, nxt).start()

                gather_desc(blk, cur).wait()
                select(blk, cur, ocur)
                write_desc(blk, ocur).start()
            return carry

        lax.fori_loop(0, n_bursts // 2, step, 0)
        write_desc(n_bursts - 1, ob1).wait()

    return body


def kernel(token_ids, weight):
    orig_shape = token_ids.shape
    idx = token_ids.reshape(-1).astype(jnp.int32)
    total = idx.shape[0]
    assert total % (NUM_WORKERS * BURST) == 0
    n_bursts = total // (NUM_WORKERS * BURST)
    idx2d = idx.reshape(total // BURST, BURST)
    w128 = weight.reshape(weight.shape[0] // PACK, 128)
    out = _gather_kernel(n_bursts)(idx2d, w128)
    return out.reshape(*orig_shape, EMB_DIM)
